# baseline (speedup 1.0000x reference)
import functools

import jax
import jax.numpy as jnp
from jax import lax
from jax.experimental import pallas as pl
from jax.experimental.pallas import tpu as pltpu
from jax.experimental.pallas import tpu_sc as plsc

N = 10000
M = 10000
E = 320000
IN_C, HID_C, OUT_C = 128, 128, 64

NC, NS = 2, 16
NW = NC * NS
EPW = E // NW
CH = 100
NCH = EPW // CH
DEGW = 16
MP = 10240
RPS = MP // NS
ZR = 128

_mesh = plsc.VectorSubcoreMesh(core_axis_name="c", subcore_axis_name="s")
_sc_params = pltpu.CompilerParams(use_tc_tiling_on_sc=False,
                                 disable_bounds_checks=True,
                                 disable_semaphore_checks=True)


def _zero_fill(buf, width):
    zero16 = jnp.zeros((16,), jnp.float32)

    def row(i, carry):
        for g in range(width // 16):
            buf[i, pl.ds(g * 16, 16)] = zero16
        return carry

    lax.fori_loop(0, buf.shape[0], row, 0)


def _sc_propagate(table, sidx3, didx3, width, with_degrees=False):
    out_type = jax.ShapeDtypeStruct((NC, MP, width), jnp.float32)
    deg_type = jax.ShapeDtypeStruct((NC, MP), jnp.float32)
    if with_degrees:
        out_type = (out_type, deg_type, deg_type)
    deg_scratch = [
        pltpu.VMEM((CH,), jnp.float32),
        pltpu.VMEM_SHARED((MP,), jnp.float32),
        pltpu.VMEM_SHARED((MP,), jnp.float32),
        pltpu.SemaphoreType.DMA,
        pltpu.SemaphoreType.DMA,
    ] if with_degrees else []

    @functools.partial(
        pl.kernel,
        out_type=out_type,
        mesh=_mesh,
        compiler_params=_sc_params,
        scratch_types=[
            pltpu.VMEM((NCH, CH), jnp.int32),
            pltpu.VMEM((NCH, CH), jnp.int32),
            pltpu.VMEM((CH, width), jnp.float32),
            pltpu.VMEM((CH, width), jnp.float32),
            pltpu.VMEM_SHARED((MP, width), jnp.float32),
            pltpu.SemaphoreType.DMA,
            pltpu.SemaphoreType.DMA,
            pltpu.SemaphoreType.DMA,
            pltpu.SemaphoreType.DMA,
        ] + deg_scratch,
    )
    def k(table_hbm, sidx_hbm, didx_hbm, out_hbm, *rest):
        if with_degrees:
            (sdout, ddout, sv, dv, rows0, rows1, acc, sem0, sem1,
             ssem0, ssem1, ones_v, sdacc, ddacc, sdsem, ddsem) = rest
        else:
            sv, dv, rows0, rows1, acc, sem0, sem1, ssem0, ssem1 = rest
        cid = lax.axis_index("c")
        sid = lax.axis_index("s")
        wid = cid * NS + sid

        _zero_fill(rows0, width)
        pltpu.sync_copy(sidx_hbm.at[wid], sv)
        pltpu.sync_copy(didx_hbm.at[wid], dv)

        r0 = sid * RPS
        for t in range(RPS // CH):
            pltpu.sync_copy(rows0, acc.at[pl.ds(r0 + t * CH, CH)])
        pltpu.sync_copy(rows0.at[pl.ds(0, RPS - (RPS // CH) * CH)],
                        acc.at[pl.ds(r0 + (RPS // CH) * CH,
                                     RPS - (RPS // CH) * CH)])
        if with_degrees:
            one16 = jnp.ones((16,), jnp.float32)

            def fill_ones(i, carry):
                ones_v[pl.ds(i * 16, 16)] = one16
                return carry

            lax.fori_loop(0, CH // 16, fill_ones, 0)
            ones_v[pl.ds(CH - 16, 16)] = one16
            for t in range(RPS // width):
                pltpu.sync_copy(rows0.at[0], sdacc.at[pl.ds(r0 + t * width,
                                                            width)])
                pltpu.sync_copy(rows0.at[0], ddacc.at[pl.ds(r0 + t * width,
                                                            width)])
        plsc.subcore_barrier()

        pltpu.async_copy(table_hbm.at[sv.at[0]], rows0, sem0)
        pltpu.async_copy(table_hbm.at[sv.at[1]], rows1, sem1)

        def body(i, carry):
            j0 = 2 * i
            j1 = 2 * i + 1
            pltpu.make_async_copy(table_hbm.at[sv.at[j0]], rows0, sem0).wait()
            pltpu.async_copy(rows0, acc.at[dv.at[j0]], ssem0, add=True)
            pltpu.make_async_copy(table_hbm.at[sv.at[j1]], rows1, sem1).wait()
            pltpu.async_copy(rows1, acc.at[dv.at[j1]], ssem1, add=True)
            if with_degrees:
                sd0 = pltpu.async_copy(ones_v, sdacc.at[sv.at[j0]], sdsem,
                                       add=True)
                dd0 = pltpu.async_copy(ones_v, ddacc.at[dv.at[j0]], ddsem,
                                       add=True)
                sd1 = pltpu.async_copy(ones_v, sdacc.at[sv.at[j1]], sdsem,
                                       add=True)
                dd1 = pltpu.async_copy(ones_v, ddacc.at[dv.at[j1]], ddsem,
                                       add=True)
                sd0.wait()
                dd0.wait()
                sd1.wait()
                dd1.wait()

            @pl.when(i < NCH // 2 - 1)
            def _():
                pltpu.make_async_copy(table_hbm.at[sv.at[j0]],
                                      rows0, ssem0).wait()
                pltpu.async_copy(table_hbm.at[sv.at[j0 + 2]], rows0, sem0)
                pltpu.make_async_copy(table_hbm.at[sv.at[j1]],
                                      rows1, ssem1).wait()
                pltpu.async_copy(table_hbm.at[sv.at[j1 + 2]], rows1, sem1)
            return carry

        lax.fori_loop(0, NCH // 2, body, 0)
        pltpu.make_async_copy(table_hbm.at[sv.at[0]], rows0, ssem0).wait()
        pltpu.make_async_copy(table_hbm.at[sv.at[1]], rows1, ssem1).wait()
        plsc.subcore_barrier()

        pltpu.sync_copy(acc.at[pl.ds(r0, RPS)], out_hbm.at[cid, pl.ds(r0, RPS)])
        if with_degrees:
            pltpu.sync_copy(sdacc.at[pl.ds(r0, RPS)],
                            sdout.at[cid, pl.ds(r0, RPS)])
            pltpu.sync_copy(ddacc.at[pl.ds(r0, RPS)],
                            ddout.at[cid, pl.ds(r0, RPS)])

    return k(table, sidx3, didx3)


def _tc_matmul1(x, W1):
    def body(x_ref, w_ref, o_ref):
        o_ref[...] = lax.dot_general(
            x_ref[...], w_ref[...], (((1,), (1,)), ((), ())),
            preferred_element_type=jnp.float32)

    return pl.pallas_call(
        body,
        grid=(10,),
        in_specs=[
            pl.BlockSpec((1000, IN_C), lambda i: (i, 0)),
            pl.BlockSpec((HID_C, IN_C), lambda i: (0, 0)),
        ],
        out_specs=pl.BlockSpec((1000, HID_C), lambda i: (i, 0)),
        out_shape=jax.ShapeDtypeStruct((N, HID_C), jnp.float32),
    )(x, W1)


def _tc_combine_scale(p0, p1, g0, g1, width):
    def body(a_ref, b_ref, g0_ref, g1_ref, o_ref):
        deg = g0_ref[...] + g1_ref[...]
        inv = jnp.where(deg > 0, 1.0 / deg, 0.0)
        o_ref[...] = (a_ref[...] + b_ref[...]) * inv

    return pl.pallas_call(
        body,
        grid=(10,),
        in_specs=[
            pl.BlockSpec((1000, width), lambda i: (i, 0)),
            pl.BlockSpec((1000, width), lambda i: (i, 0)),
            pl.BlockSpec((1000, 1), lambda i: (i, 0)),
            pl.BlockSpec((1000, 1), lambda i: (i, 0)),
        ],
        out_specs=pl.BlockSpec((1000, width), lambda i: (i, 0)),
        out_shape=jax.ShapeDtypeStruct((M, width), jnp.float32),
    )(p0, p1, g0, g1)


def _tc_scale_bias_relu_matmul(p0, p1, g0, g1, b1, W2):
    def body(a_ref, b_ref, g0_ref, g1_ref, bias_ref, w_ref, o_ref):
        deg = g0_ref[...] + g1_ref[...]
        inv = jnp.where(deg > 0, 1.0 / deg, 0.0)
        h = (a_ref[...] + b_ref[...]) * inv + bias_ref[...]
        h = jnp.maximum(h, 0.0)
        o_ref[...] = lax.dot_general(
            h, w_ref[...], (((1,), (1,)), ((), ())),
            preferred_element_type=jnp.float32)

    return pl.pallas_call(
        body,
        grid=(10,),
        in_specs=[
            pl.BlockSpec((1000, HID_C), lambda i: (i, 0)),
            pl.BlockSpec((1000, HID_C), lambda i: (i, 0)),
            pl.BlockSpec((1000, 1), lambda i: (i, 0)),
            pl.BlockSpec((1000, 1), lambda i: (i, 0)),
            pl.BlockSpec((1, HID_C), lambda i: (0, 0)),
            pl.BlockSpec((OUT_C, HID_C), lambda i: (0, 0)),
        ],
        out_specs=pl.BlockSpec((1000, OUT_C), lambda i: (i, 0)),
        out_shape=jax.ShapeDtypeStruct((N, OUT_C), jnp.float32),
    )(p0, p1, g0, g1, b1, W2)


def _tc_scale_bias_logsoftmax(p0, p1, g0, g1, b2):
    def body(a_ref, b_ref, g0_ref, g1_ref, bias_ref, o_ref):
        deg = g0_ref[...] + g1_ref[...]
        inv = jnp.where(deg > 0, 1.0 / deg, 0.0)
        z = (a_ref[...] + b_ref[...]) * inv + bias_ref[...]
        m = jnp.max(z, axis=1, keepdims=True)
        e = jnp.exp(z - m)
        s = jnp.sum(e, axis=1, keepdims=True)
        o_ref[...] = z - m - jnp.log(s)

    return pl.pallas_call(
        body,
        grid=(10,),
        in_specs=[
            pl.BlockSpec((1000, OUT_C), lambda i: (i, 0)),
            pl.BlockSpec((1000, OUT_C), lambda i: (i, 0)),
            pl.BlockSpec((1000, 1), lambda i: (i, 0)),
            pl.BlockSpec((1000, 1), lambda i: (i, 0)),
            pl.BlockSpec((1, OUT_C), lambda i: (0, 0)),
        ],
        out_specs=pl.BlockSpec((1000, OUT_C), lambda i: (i, 0)),
        out_shape=jax.ShapeDtypeStruct((N, OUT_C), jnp.float32),
    )(p0, p1, g0, g1, b2)


@jax.jit
def kernel(x, edge_index, W1, b1, W2, b2):
    node_idx3 = edge_index[0].reshape(NW, NCH, CH)
    he_idx3 = edge_index[1].reshape(NW, NCH, CH)
    b1r = b1.reshape(1, HID_C)
    b2r = b2.reshape(1, OUT_C)

    y1 = _tc_matmul1(x, W1)

    hp, dpart, bpart = _sc_propagate(y1, node_idx3, he_idx3, HID_C,
                                     with_degrees=True)
    d0, d1 = dpart[0].reshape(MP, 1), dpart[1].reshape(MP, 1)
    bd0, bd1 = bpart[0].reshape(MP, 1), bpart[1].reshape(MP, 1)
    he1 = _tc_combine_scale(hp[0], hp[1], bd0, bd1, HID_C)

    np_ = _sc_propagate(he1, he_idx3, node_idx3, HID_C)
    y2 = _tc_scale_bias_relu_matmul(np_[0], np_[1], d0, d1, b1r, W2)

    hp2 = _sc_propagate(y2, node_idx3, he_idx3, OUT_C)
    he2 = _tc_combine_scale(hp2[0], hp2[1], bd0, bd1, OUT_C)

    np2 = _sc_propagate(he2, he_idx3, node_idx3, OUT_C)
    return _tc_scale_bias_logsoftmax(np2[0], np2[1], d0, d1, b2r)


# speedup vs baseline: 1.0153x; 1.0153x over previous
import functools

import jax
import jax.numpy as jnp
from jax import lax
from jax.experimental import pallas as pl
from jax.experimental.pallas import tpu as pltpu
from jax.experimental.pallas import tpu_sc as plsc

N = 10000
M = 10000
E = 320000
IN_C, HID_C, OUT_C = 128, 128, 64

NC, NS = 2, 16
NW = NC * NS
EPW = E // NW
CH = 100
NCH = EPW // CH
DEGW = 16
MP = 10240
RPS = MP // NS
ZR = 128

_mesh = plsc.VectorSubcoreMesh(core_axis_name="c", subcore_axis_name="s")
_sc_params = pltpu.CompilerParams(use_tc_tiling_on_sc=False,
                                 disable_bounds_checks=True,
                                 disable_semaphore_checks=True)


def _zero_fill(buf, width):
    zero16 = jnp.zeros((16,), jnp.float32)

    def row(i, carry):
        for g in range(width // 16):
            buf[i, pl.ds(g * 16, 16)] = zero16
        return carry

    lax.fori_loop(0, buf.shape[0], row, 0)


def _sc_propagate(table, sidx3, didx3, width, with_degrees=False):
    out_type = jax.ShapeDtypeStruct((NC, MP, width), jnp.float32)
    deg_type = jax.ShapeDtypeStruct((NC, MP), jnp.float32)
    if with_degrees:
        out_type = (out_type, deg_type, deg_type)
    deg_scratch = [
        pltpu.VMEM((CH,), jnp.float32),
        pltpu.VMEM_SHARED((MP,), jnp.float32),
        pltpu.VMEM_SHARED((MP,), jnp.float32),
        pltpu.SemaphoreType.DMA,
        pltpu.SemaphoreType.DMA,
    ] if with_degrees else []

    @functools.partial(
        pl.kernel,
        out_type=out_type,
        mesh=_mesh,
        compiler_params=_sc_params,
        scratch_types=[
            pltpu.VMEM((NCH, CH), jnp.int32),
            pltpu.VMEM((NCH, CH), jnp.int32),
            pltpu.VMEM((CH, width), jnp.float32),
            pltpu.VMEM((CH, width), jnp.float32),
            pltpu.VMEM_SHARED((MP, width), jnp.float32),
            pltpu.SemaphoreType.DMA,
            pltpu.SemaphoreType.DMA,
            pltpu.SemaphoreType.DMA,
            pltpu.SemaphoreType.DMA,
        ] + deg_scratch,
    )
    def k(table_hbm, sidx_hbm, didx_hbm, out_hbm, *rest):
        if with_degrees:
            (sdout, ddout, sv, dv, rows0, rows1, acc, sem0, sem1,
             ssem0, ssem1, ones_v, sdacc, ddacc, sdsem, ddsem) = rest
        else:
            sv, dv, rows0, rows1, acc, sem0, sem1, ssem0, ssem1 = rest
        cid = lax.axis_index("c")
        sid = lax.axis_index("s")
        wid = cid * NS + sid

        _zero_fill(rows0, width)
        pltpu.sync_copy(sidx_hbm.at[wid], sv)
        pltpu.sync_copy(didx_hbm.at[wid], dv)

        r0 = sid * RPS
        for t in range(RPS // CH):
            pltpu.sync_copy(rows0, acc.at[pl.ds(r0 + t * CH, CH)])
        pltpu.sync_copy(rows0.at[pl.ds(0, RPS - (RPS // CH) * CH)],
                        acc.at[pl.ds(r0 + (RPS // CH) * CH,
                                     RPS - (RPS // CH) * CH)])
        if with_degrees:
            one16 = jnp.ones((16,), jnp.float32)

            def fill_ones(i, carry):
                ones_v[pl.ds(i * 16, 16)] = one16
                return carry

            lax.fori_loop(0, CH // 16, fill_ones, 0)
            ones_v[pl.ds(CH - 16, 16)] = one16
            for t in range(RPS // width):
                pltpu.sync_copy(rows0.at[0], sdacc.at[pl.ds(r0 + t * width,
                                                            width)])
                pltpu.sync_copy(rows0.at[0], ddacc.at[pl.ds(r0 + t * width,
                                                            width)])
        plsc.subcore_barrier()

        pltpu.async_copy(table_hbm.at[sv.at[0]], rows0, sem0)
        pltpu.async_copy(table_hbm.at[sv.at[1]], rows1, sem1)

        def body(i, carry):
            j0 = 2 * i
            j1 = 2 * i + 1
            pltpu.make_async_copy(table_hbm.at[sv.at[j0]], rows0, sem0).wait()
            pltpu.async_copy(rows0, acc.at[dv.at[j0]], ssem0, add=True)
            pltpu.make_async_copy(table_hbm.at[sv.at[j1]], rows1, sem1).wait()
            pltpu.async_copy(rows1, acc.at[dv.at[j1]], ssem1, add=True)
            if with_degrees:
                pltpu.async_copy(ones_v, sdacc.at[sv.at[j0]], sdsem, add=True)
                pltpu.async_copy(ones_v, ddacc.at[dv.at[j0]], ddsem, add=True)
                pltpu.async_copy(ones_v, sdacc.at[sv.at[j1]], sdsem, add=True)
                pltpu.async_copy(ones_v, ddacc.at[dv.at[j1]], ddsem, add=True)

            @pl.when(i < NCH // 2 - 1)
            def _():
                pltpu.make_async_copy(table_hbm.at[sv.at[j0]],
                                      rows0, ssem0).wait()
                pltpu.async_copy(table_hbm.at[sv.at[j0 + 2]], rows0, sem0)
                pltpu.make_async_copy(table_hbm.at[sv.at[j1]],
                                      rows1, ssem1).wait()
                pltpu.async_copy(table_hbm.at[sv.at[j1 + 2]], rows1, sem1)
            return carry

        lax.fori_loop(0, NCH // 2, body, 0)
        pltpu.make_async_copy(table_hbm.at[sv.at[0]], rows0, ssem0).wait()
        pltpu.make_async_copy(table_hbm.at[sv.at[1]], rows1, ssem1).wait()
        if with_degrees:

            def drain(j, carry):
                pltpu.make_async_copy(ones_v, sdacc.at[sv.at[0]],
                                      sdsem).wait()
                pltpu.make_async_copy(ones_v, ddacc.at[dv.at[0]],
                                      ddsem).wait()
                return carry

            lax.fori_loop(0, NCH, drain, 0)
        plsc.subcore_barrier()

        pltpu.sync_copy(acc.at[pl.ds(r0, RPS)], out_hbm.at[cid, pl.ds(r0, RPS)])
        if with_degrees:
            pltpu.sync_copy(sdacc.at[pl.ds(r0, RPS)],
                            sdout.at[cid, pl.ds(r0, RPS)])
            pltpu.sync_copy(ddacc.at[pl.ds(r0, RPS)],
                            ddout.at[cid, pl.ds(r0, RPS)])

    return k(table, sidx3, didx3)


def _tc_matmul1(x, W1):
    def body(x_ref, w_ref, o_ref):
        o_ref[...] = lax.dot_general(
            x_ref[...], w_ref[...], (((1,), (1,)), ((), ())),
            preferred_element_type=jnp.float32)

    return pl.pallas_call(
        body,
        grid=(10,),
        in_specs=[
            pl.BlockSpec((1000, IN_C), lambda i: (i, 0)),
            pl.BlockSpec((HID_C, IN_C), lambda i: (0, 0)),
        ],
        out_specs=pl.BlockSpec((1000, HID_C), lambda i: (i, 0)),
        out_shape=jax.ShapeDtypeStruct((N, HID_C), jnp.float32),
    )(x, W1)


def _tc_combine_scale(p0, p1, g0, g1, width):
    def body(a_ref, b_ref, g0_ref, g1_ref, o_ref):
        deg = g0_ref[...] + g1_ref[...]
        inv = jnp.where(deg > 0, 1.0 / deg, 0.0)
        o_ref[...] = (a_ref[...] + b_ref[...]) * inv

    return pl.pallas_call(
        body,
        grid=(10,),
        in_specs=[
            pl.BlockSpec((1000, width), lambda i: (i, 0)),
            pl.BlockSpec((1000, width), lambda i: (i, 0)),
            pl.BlockSpec((1000, 1), lambda i: (i, 0)),
            pl.BlockSpec((1000, 1), lambda i: (i, 0)),
        ],
        out_specs=pl.BlockSpec((1000, width), lambda i: (i, 0)),
        out_shape=jax.ShapeDtypeStruct((M, width), jnp.float32),
    )(p0, p1, g0, g1)


def _tc_scale_bias_relu_matmul(p0, p1, g0, g1, b1, W2):
    def body(a_ref, b_ref, g0_ref, g1_ref, bias_ref, w_ref, o_ref):
        deg = g0_ref[...] + g1_ref[...]
        inv = jnp.where(deg > 0, 1.0 / deg, 0.0)
        h = (a_ref[...] + b_ref[...]) * inv + bias_ref[...]
        h = jnp.maximum(h, 0.0)
        o_ref[...] = lax.dot_general(
            h, w_ref[...], (((1,), (1,)), ((), ())),
            preferred_element_type=jnp.float32)

    return pl.pallas_call(
        body,
        grid=(10,),
        in_specs=[
            pl.BlockSpec((1000, HID_C), lambda i: (i, 0)),
            pl.BlockSpec((1000, HID_C), lambda i: (i, 0)),
            pl.BlockSpec((1000, 1), lambda i: (i, 0)),
            pl.BlockSpec((1000, 1), lambda i: (i, 0)),
            pl.BlockSpec((1, HID_C), lambda i: (0, 0)),
            pl.BlockSpec((OUT_C, HID_C), lambda i: (0, 0)),
        ],
        out_specs=pl.BlockSpec((1000, OUT_C), lambda i: (i, 0)),
        out_shape=jax.ShapeDtypeStruct((N, OUT_C), jnp.float32),
    )(p0, p1, g0, g1, b1, W2)


def _tc_scale_bias_logsoftmax(p0, p1, g0, g1, b2):
    def body(a_ref, b_ref, g0_ref, g1_ref, bias_ref, o_ref):
        deg = g0_ref[...] + g1_ref[...]
        inv = jnp.where(deg > 0, 1.0 / deg, 0.0)
        z = (a_ref[...] + b_ref[...]) * inv + bias_ref[...]
        m = jnp.max(z, axis=1, keepdims=True)
        e = jnp.exp(z - m)
        s = jnp.sum(e, axis=1, keepdims=True)
        o_ref[...] = z - m - jnp.log(s)

    return pl.pallas_call(
        body,
        grid=(10,),
        in_specs=[
            pl.BlockSpec((1000, OUT_C), lambda i: (i, 0)),
            pl.BlockSpec((1000, OUT_C), lambda i: (i, 0)),
            pl.BlockSpec((1000, 1), lambda i: (i, 0)),
            pl.BlockSpec((1000, 1), lambda i: (i, 0)),
            pl.BlockSpec((1, OUT_C), lambda i: (0, 0)),
        ],
        out_specs=pl.BlockSpec((1000, OUT_C), lambda i: (i, 0)),
        out_shape=jax.ShapeDtypeStruct((N, OUT_C), jnp.float32),
    )(p0, p1, g0, g1, b2)


@jax.jit
def kernel(x, edge_index, W1, b1, W2, b2):
    node_idx3 = edge_index[0].reshape(NW, NCH, CH)
    he_idx3 = edge_index[1].reshape(NW, NCH, CH)
    b1r = b1.reshape(1, HID_C)
    b2r = b2.reshape(1, OUT_C)

    y1 = _tc_matmul1(x, W1)

    hp, dpart, bpart = _sc_propagate(y1, node_idx3, he_idx3, HID_C,
                                     with_degrees=True)
    d0, d1 = dpart[0].reshape(MP, 1), dpart[1].reshape(MP, 1)
    bd0, bd1 = bpart[0].reshape(MP, 1), bpart[1].reshape(MP, 1)
    he1 = _tc_combine_scale(hp[0], hp[1], bd0, bd1, HID_C)

    np_ = _sc_propagate(he1, he_idx3, node_idx3, HID_C)
    y2 = _tc_scale_bias_relu_matmul(np_[0], np_[1], d0, d1, b1r, W2)

    hp2 = _sc_propagate(y2, node_idx3, he_idx3, OUT_C)
    he2 = _tc_combine_scale(hp2[0], hp2[1], bd0, bd1, OUT_C)

    np2 = _sc_propagate(he2, he_idx3, node_idx3, OUT_C)
    return _tc_scale_bias_logsoftmax(np2[0], np2[1], d0, d1, b2r)
